# baseline (speedup 1.0000x reference)
import functools

import jax
import jax.numpy as jnp
from jax import lax
from jax.experimental import pallas as pl
from jax.experimental.pallas import tpu as pltpu
from jax.experimental.pallas import tpu_sc as plsc

_NC = 2
_NS = 16
_NW = _NC * _NS
_CHUNK = 128


def _sc_gather_body(nchunks, bpw, ids_hbm, z_hbm, d_hbm, zout_hbm, dout_hbm,
                    idx_v, zrows, drows, sem):
    wid = lax.axis_index("s") * _NC + lax.axis_index("c")
    base = wid * bpw
    pltpu.sync_copy(ids_hbm.at[pl.ds(wid * nchunks, nchunks)], idx_v)
    copies = []
    for j in range(nchunks):
        copies.append(pltpu.async_copy(
            z_hbm.at[idx_v.at[j]], zrows.at[pl.ds(j * _CHUNK, _CHUNK)], sem))
        copies.append(pltpu.async_copy(
            d_hbm.at[idx_v.at[j]], drows.at[pl.ds(j * _CHUNK, _CHUNK)], sem))
    for cp in copies:
        cp.wait()
    pltpu.sync_copy(zrows, zout_hbm.at[pl.ds(base, bpw)])
    pltpu.sync_copy(drows, dout_hbm.at[pl.ds(base, bpw)])


def _sc_gather(bin_ids, latent_z, latent_d):
    b = bin_ids.shape[0]
    zd = latent_z.shape[1]
    ed = latent_d.shape[1]
    bpw = b // _NW
    nchunks = bpw // _CHUNK
    ids2d = bin_ids.reshape(b // _CHUNK, _CHUNK)
    mesh = plsc.VectorSubcoreMesh(core_axis_name="c", subcore_axis_name="s")
    run = pl.kernel(
        functools.partial(_sc_gather_body, nchunks, bpw),
        out_type=(
            jax.ShapeDtypeStruct((b, zd), jnp.float32),
            jax.ShapeDtypeStruct((b, ed), jnp.float32),
        ),
        mesh=mesh,
        scratch_types=[
            pltpu.VMEM((nchunks, _CHUNK), jnp.int32),
            pltpu.VMEM((bpw, zd), jnp.float32),
            pltpu.VMEM((bpw, ed), jnp.float32),
            pltpu.SemaphoreType.DMA,
        ],
        compiler_params=pltpu.CompilerParams(use_tc_tiling_on_sc=False),
    )
    return run(ids2d, latent_z, latent_d)


def _tc_mlp_body(x_ref, z_ref, d_ref, w1x_ref, w1z_ref, b1_ref, w2_ref,
                 b2_ref, wf_ref, out_ref):
    h = x_ref[...] @ w1x_ref[...] + z_ref[...] @ w1z_ref[...] + b1_ref[...]
    h = jnp.maximum(h, 0.0)
    intrinsic = h @ w2_ref[...] + b2_ref[...]
    gated = jax.nn.sigmoid(d_ref[...])
    out_ref[...] = jnp.sum(intrinsic * gated * wf_ref[...], axis=1)[None, None, :]


def _tc_mlp(x, z_rows, d_rows, W1, b1, W2, b2, Wf):
    b, xd = x.shape
    zd = z_rows.shape[1]
    ed = d_rows.shape[1]
    hd = W1.shape[1]
    bm = 2048
    grid = b // bm
    w1x = W1[:xd]
    w1z = W1[xd:]
    out = pl.pallas_call(
        _tc_mlp_body,
        grid=(grid,),
        in_specs=[
            pl.BlockSpec((bm, xd), lambda i: (i, 0)),
            pl.BlockSpec((bm, zd), lambda i: (i, 0)),
            pl.BlockSpec((bm, ed), lambda i: (i, 0)),
            pl.BlockSpec((xd, hd), lambda i: (0, 0)),
            pl.BlockSpec((zd, hd), lambda i: (0, 0)),
            pl.BlockSpec((1, hd), lambda i: (0, 0)),
            pl.BlockSpec((hd, ed), lambda i: (0, 0)),
            pl.BlockSpec((1, ed), lambda i: (0, 0)),
            pl.BlockSpec((1, ed), lambda i: (0, 0)),
        ],
        out_specs=pl.BlockSpec((1, 1, bm), lambda i: (i, 0, 0)),
        out_shape=jax.ShapeDtypeStruct((grid, 1, bm), jnp.float32),
    )(x, z_rows, d_rows, w1x, w1z, b1.reshape(1, hd), W2,
      b2.reshape(1, ed), Wf.reshape(1, ed))
    return out.reshape(b)


def kernel(x, bin_ids, latent_z, latent_d, W1, b1, W2, b2, Wf):
    z_rows, d_rows = _sc_gather(bin_ids, latent_z, latent_d)
    return _tc_mlp(x, z_rows, d_rows, W1, b1, W2, b2, Wf)
